# baseline (speedup 1.0000x reference)
import functools
import math

import jax
import jax.numpy as jnp
from jax import lax
from jax.experimental import pallas as pl
from jax.experimental.pallas import tpu as pltpu
from jax.experimental.pallas import tpu_sc as plsc

B, J, O, M = 128, 40, 25, 64
D = 256
SCALE = 100.0
JO = J * O
OP = 32
JOP = J * OP
P = 64
C_AVG, C_NEL, C_RDY = 61, 62, 63
NB = 4
R = NB * JOP


_SC_CORES = 2
_SC_SUBCORES = 16
_NW = _SC_CORES * _SC_SUBCORES
_BPW = B // _NW


_JC = 8


def _edge_sc_kernel(pt_hbm, out_hbm, buf_in, buf_out):
    wid = lax.axis_index("s") * _SC_CORES + lax.axis_index("c")

    def body_b(t, carry):
        b = wid * _BPW + t

        def body_c(c, c1):
            j0 = pl.multiple_of(c * _JC, _JC)
            pltpu.sync_copy(pt_hbm.at[b, pl.ds(j0, _JC)], buf_in)

            def body_r(r, c2):
                for jj in range(_JC):
                    for k in range(M // 16):
                        sl = pl.ds(k * 16, 16)
                        buf_out[jj * O + r, sl] = buf_in[jj, r, sl] * (1.0 / SCALE)
                return c2
            lax.fori_loop(0, O, body_r, 0, unroll=False)
            row0 = pl.multiple_of(c * (_JC * O), 8)
            pltpu.sync_copy(buf_out, out_hbm.at[b, pl.ds(row0, _JC * O), :])
            return c1
        lax.fori_loop(0, J // _JC, body_c, 0, unroll=False)
        return carry

    lax.fori_loop(0, _BPW, body_b, 0, unroll=False)


_edge_sc = functools.partial(
    pl.kernel,
    mesh=plsc.VectorSubcoreMesh(core_axis_name="c", subcore_axis_name="s"),
    out_type=jax.ShapeDtypeStruct((B, JO, M), jnp.float32),
    scratch_types=[pltpu.VMEM((_JC, O, M), jnp.float32),
                   pltpu.VMEM((_JC * O, M), jnp.float32)],
)(_edge_sc_kernel)


def _fused_kernel(pt_ref, no_ref, tjr_ref, jd_ref, tmr_ref, rem_ref,
                  wops_ref, wma_ref, ops_ref, ma_ref, tab_ref, j1h_ref):
    f32 = jnp.float32
    b = pl.program_id(0)

    @pl.when(b == 0)
    def _init():
        p_i = lax.broadcasted_iota(jnp.int32, (P, D), 0)
        d_i = lax.broadcasted_iota(jnp.int32, (P, D), 1)
        d_par = (d_i & 1).astype(f32)
        d_even = (d_i - (d_i & 1)).astype(f32)
        ang = (p_i.astype(f32) * jnp.exp(d_even * (-math.log(10000.0) / D))
               + d_par * (math.pi / 2.0))
        pe = jnp.sin(ang)
        w = wops_ref[...]
        pe = jnp.where(p_i == C_AVG, w[0:1, :], pe)
        pe = jnp.where(p_i == C_NEL, w[1:2, :], pe)
        pe = jnp.where(p_i == C_RDY, w[2:3, :], pe)
        tab_ref[...] = pe
        r1 = lax.broadcasted_iota(jnp.int32, (JOP, 1), 0)
        j1h_ref[...] = (lax.broadcasted_iota(jnp.int32, (JOP, J), 1)
                        == (r1 >> 5)).astype(f32)

    pt = pt_ref[...]

    pt2 = jnp.concatenate(
        [pt, jnp.zeros((NB, J, OP - O, M), f32)], axis=2).reshape(R, M)
    pos_mask = (pt2 > 0.0).astype(f32)
    ones_avg = jnp.full((M, 1), 1.0 / (M * SCALE), f32)
    ones_nel = jnp.full((M, 1), 1.0 / M, f32)
    avg = jnp.dot(pt2, ones_avg, preferred_element_type=f32)
    nelig = jnp.dot(pos_mask, ones_nel, preferred_element_type=f32)

    r = lax.broadcasted_iota(jnp.int32, (R, 1), 0)
    o_row = (r & (OP - 1)).astype(f32)
    j1h = j1h_ref[...]
    nr_parts = []
    for n in range(NB):
        no_n = no_ref[pl.ds(b * NB + n, 1), :]
        tjr_n = tjr_ref[pl.ds(b * NB + n, 1), :]
        jd_n = jd_ref[pl.ds(b * NB + n, 1), :]
        sched_n = jnp.where(jd_n > 0.0, 0.0, tjr_n - jnp.min(tjr_n))
        nr_parts.append(lax.dot_general(
            j1h, jnp.concatenate([no_n, sched_n], axis=0),
            (((1,), (1,)), ((), ())), preferred_element_type=f32))
    nr = jnp.concatenate(nr_parts, axis=0)
    no_r = nr[:, 0:1]
    sched_r = nr[:, 1:2]
    opready = jnp.where(o_row == no_r, sched_r, 0.0) * (1.0 / SCALE)

    pos = o_row + no_r
    l_i = lax.broadcasted_iota(jnp.int32, (R, P), 1)
    g = jnp.where(l_i == C_AVG, avg,
                  jnp.where(l_i == C_NEL, nelig,
                            jnp.where(l_i == C_RDY, opready,
                                      (pos == l_i.astype(f32)).astype(f32))))
    ops32 = jnp.dot(g, tab_ref[...], preferred_element_type=f32)
    ops_ref[...] = ops32.reshape(NB * J, OP, D)[:, :O, :] \
        .reshape(NB, J, O, D)

    nem = jnp.sum(pos_mask.reshape(NB, JOP, M), axis=1)
    for n in range(NB):
        tmr_n = tmr_ref[pl.ds(b * NB + n, 1), :]
        a_ma_n = (tmr_n - jnp.min(tmr_n)) * (1.0 / SCALE)
        rem_n = jnp.sum(rem_ref[pl.ds(b * NB + n, 1), :])
        frac_n = nem[n:n + 1, :] * (1.0 / (rem_n + 1e-6))
        mam = jnp.concatenate([a_ma_n, frac_n], axis=0)
        ma_ref[n] = lax.dot_general(mam, wma_ref[...],
                                    (((0,), (0,)), ((), ())),
                                    preferred_element_type=f32)


@functools.partial(jax.jit, static_argnames=())
def kernel(proc_times, next_op, time_job_ready, job_done, time_ma_ready,
           pad_mask, op_scheduled, W_ops, W_ma):
    f32 = jnp.float32
    no_f = next_op.astype(f32)
    jd_f = job_done.astype(f32)
    rem_f = jnp.logical_not(jnp.logical_or(pad_mask, op_scheduled)) \
        .astype(f32).reshape(B, JO)
    wopsT = W_ops.T
    wmaT = W_ma.T

    full = lambda shape: pl.BlockSpec(shape, lambda b: (0,) * len(shape))

    edge = _edge_sc(proc_times)

    ops, ma = pl.pallas_call(
        _fused_kernel,
        grid=(B // NB,),
        in_specs=[
            pl.BlockSpec((NB, J, O, M), lambda b: (b, 0, 0, 0)),
            full((B, J)),
            full((B, J)),
            full((B, J)),
            full((B, M)),
            full((B, JO)),
            full((3, D)),
            full((2, D)),
        ],
        out_specs=[
            pl.BlockSpec((NB, J, O, D), lambda b: (b, 0, 0, 0)),
            pl.BlockSpec((NB, M, D), lambda b: (b, 0, 0)),
        ],
        out_shape=[
            jax.ShapeDtypeStruct((B, J, O, D), f32),
            jax.ShapeDtypeStruct((B, M, D), f32),
        ],
        scratch_shapes=[
            pltpu.VMEM((P, D), f32),
            pltpu.VMEM((JOP, J), f32),
        ],
    )(proc_times, no_f, time_job_ready, jd_f, time_ma_ready, rem_f, wopsT, wmaT)

    return ops, ma, edge


# speedup vs baseline: 1.0822x; 1.0822x over previous
import functools
import math

import jax
import jax.numpy as jnp
from jax import lax
from jax.experimental import pallas as pl
from jax.experimental.pallas import tpu as pltpu

B, J, O, M = 128, 40, 25, 64
D = 256
SCALE = 100.0
JO = J * O
OP = 32
JOP = J * OP
P = 64
C_AVG, C_NEL, C_RDY = 61, 62, 63
NB = 8
R = NB * JOP


def _fused_kernel(pt_ref, no_ref, tjr_ref, jd_ref, tmr_ref, rem_ref,
                  wops_ref, wma_ref, ops_ref, ma_ref, tab_ref, j1h_ref):
    f32 = jnp.float32
    b = pl.program_id(0)

    @pl.when(b == 0)
    def _init():
        p_i = lax.broadcasted_iota(jnp.int32, (P, D), 0)
        d_i = lax.broadcasted_iota(jnp.int32, (P, D), 1)
        d_par = (d_i & 1).astype(f32)
        d_even = (d_i - (d_i & 1)).astype(f32)
        ang = (p_i.astype(f32) * jnp.exp(d_even * (-math.log(10000.0) / D))
               + d_par * (math.pi / 2.0))
        pe = jnp.sin(ang)
        w = wops_ref[...]
        pe = jnp.where(p_i == C_AVG, w[0:1, :], pe)
        pe = jnp.where(p_i == C_NEL, w[1:2, :], pe)
        pe = jnp.where(p_i == C_RDY, w[2:3, :], pe)
        tab_ref[...] = pe
        r1 = lax.broadcasted_iota(jnp.int32, (JOP, 1), 0)
        j1h_ref[...] = (lax.broadcasted_iota(jnp.int32, (JOP, J), 1)
                        == (r1 >> 5)).astype(f32)

    r1 = lax.broadcasted_iota(jnp.int32, (JOP, 1), 0)
    o_row = (r1 & (OP - 1)).astype(f32)
    l_i = lax.broadcasted_iota(jnp.int32, (JOP, P), 1)
    l_f = l_i.astype(f32)
    ones_avg = jnp.full((M, 1), 1.0 / (M * SCALE), f32)
    ones_nel = jnp.full((M, 1), 1.0 / M, f32)
    j1h = j1h_ref[...]

    for n in range(NB):
        ptn = pt_ref[n]
        pt2 = jnp.concatenate(
            [ptn, jnp.zeros((J, OP - O, M), f32)], axis=1).reshape(JOP, M)
        pos_mask = (pt2 > 0.0).astype(f32)
        avg = jnp.dot(pt2, ones_avg, preferred_element_type=f32)
        nelig = jnp.dot(pos_mask, ones_nel, preferred_element_type=f32)

        no_n = no_ref[pl.ds(b * NB + n, 1), :]
        tjr_n = tjr_ref[pl.ds(b * NB + n, 1), :]
        jd_n = jd_ref[pl.ds(b * NB + n, 1), :]
        sched_n = jnp.where(jd_n > 0.0, 0.0, tjr_n - jnp.min(tjr_n))
        nr = lax.dot_general(
            j1h, jnp.concatenate([no_n, sched_n], axis=0),
            (((1,), (1,)), ((), ())), preferred_element_type=f32)
        no_r = nr[:, 0:1]
        sched_r = nr[:, 1:2]
        opready = jnp.where(o_row == no_r, sched_r, 0.0) * (1.0 / SCALE)

        pos = o_row + no_r
        g = jnp.where(l_i == C_AVG, avg,
                      jnp.where(l_i == C_NEL, nelig,
                                jnp.where(l_i == C_RDY, opready,
                                          (pos == l_f).astype(f32))))
        ops32 = jnp.dot(g, tab_ref[...], preferred_element_type=f32)
        ops_ref[n] = ops32.reshape(J, OP, D)[:, :O, :]

        nem_n = jnp.sum(pos_mask, axis=0, keepdims=True)
        tmr_n = tmr_ref[pl.ds(b * NB + n, 1), :]
        a_ma_n = (tmr_n - jnp.min(tmr_n)) * (1.0 / SCALE)
        rem_n = jnp.sum(rem_ref[pl.ds(b * NB + n, 1), :])
        frac_n = nem_n * (1.0 / (rem_n + 1e-6))
        mam = jnp.concatenate([a_ma_n, frac_n], axis=0)
        ma_ref[n] = lax.dot_general(mam, wma_ref[...],
                                    (((0,), (0,)), ((), ())),
                                    preferred_element_type=f32)


@functools.partial(jax.jit, static_argnames=())
def kernel(proc_times, next_op, time_job_ready, job_done, time_ma_ready,
           pad_mask, op_scheduled, W_ops, W_ma):
    f32 = jnp.float32
    no_f = next_op.astype(f32)
    jd_f = job_done.astype(f32)
    rem_f = jnp.logical_not(jnp.logical_or(pad_mask, op_scheduled)) \
        .astype(f32).reshape(B, JO)
    wopsT = W_ops.T
    wmaT = W_ma.T

    full = lambda shape: pl.BlockSpec(shape, lambda b: (0,) * len(shape))

    ops, ma = pl.pallas_call(
        _fused_kernel,
        grid=(B // NB,),
        in_specs=[
            pl.BlockSpec((NB, J, O, M), lambda b: (b, 0, 0, 0)),
            full((B, J)),
            full((B, J)),
            full((B, J)),
            full((B, M)),
            full((B, JO)),
            full((3, D)),
            full((2, D)),
        ],
        out_specs=[
            pl.BlockSpec((NB, J, O, D), lambda b: (b, 0, 0, 0)),
            pl.BlockSpec((NB, M, D), lambda b: (b, 0, 0)),
        ],
        out_shape=[
            jax.ShapeDtypeStruct((B, J, O, D), f32),
            jax.ShapeDtypeStruct((B, M, D), f32),
        ],
        scratch_shapes=[
            pltpu.VMEM((P, D), f32),
            pltpu.VMEM((JOP, J), f32),
        ],
    )(proc_times, no_f, time_job_ready, jd_f, time_ma_ready, rem_f, wopsT, wmaT)

    edge = (proc_times * (1.0 / SCALE)).reshape(B, JO, M)
    return ops, ma, edge


# speedup vs baseline: 1.0898x; 1.0070x over previous
import functools
import math

import jax
import jax.numpy as jnp
from jax import lax
from jax.experimental import pallas as pl
from jax.experimental.pallas import tpu as pltpu

B, J, O, M = 128, 40, 25, 64
D = 256
SCALE = 100.0
JO = J * O
OP = 32
JOP = J * OP
P = 64
C_AVG, C_NEL, C_RDY = 61, 62, 63
NB = 4
R = NB * JOP


def _fused_kernel(pt_ref, no_ref, tjr_ref, jd_ref, tmr_ref, rem_ref,
                  wops_ref, wma_ref, ops_ref, ma_ref, tab_ref, j1h_ref):
    f32 = jnp.float32
    b = pl.program_id(0)

    @pl.when(b == 0)
    def _init():
        p_i = lax.broadcasted_iota(jnp.int32, (P, D), 0)
        d_i = lax.broadcasted_iota(jnp.int32, (P, D), 1)
        d_par = (d_i & 1).astype(f32)
        d_even = (d_i - (d_i & 1)).astype(f32)
        ang = (p_i.astype(f32) * jnp.exp(d_even * (-math.log(10000.0) / D))
               + d_par * (math.pi / 2.0))
        pe = jnp.sin(ang)
        w = wops_ref[...]
        pe = jnp.where(p_i == C_AVG, w[0:1, :], pe)
        pe = jnp.where(p_i == C_NEL, w[1:2, :], pe)
        pe = jnp.where(p_i == C_RDY, w[2:3, :], pe)
        tab_ref[...] = pe
        r1 = lax.broadcasted_iota(jnp.int32, (JOP, 1), 0)
        j1h_ref[...] = (lax.broadcasted_iota(jnp.int32, (JOP, J), 1)
                        == (r1 >> 5)).astype(f32)

    r1 = lax.broadcasted_iota(jnp.int32, (JOP, 1), 0)
    o_row = (r1 & (OP - 1)).astype(f32)
    l_i = lax.broadcasted_iota(jnp.int32, (JOP, P), 1)
    l_f = l_i.astype(f32)
    ones_avg = jnp.full((M, 1), 1.0 / (M * SCALE), f32)
    ones_nel = jnp.full((M, 1), 1.0 / M, f32)
    j1h = j1h_ref[...]

    for n in range(NB):
        ptn = pt_ref[n]
        pt2 = jnp.concatenate(
            [ptn, jnp.zeros((J, OP - O, M), f32)], axis=1).reshape(JOP, M)
        pos_mask = (pt2 > 0.0).astype(f32)
        avg = jnp.dot(pt2, ones_avg, preferred_element_type=f32)
        nelig = jnp.dot(pos_mask, ones_nel, preferred_element_type=f32)

        no_n = no_ref[pl.ds(b * NB + n, 1), :]
        tjr_n = tjr_ref[pl.ds(b * NB + n, 1), :]
        jd_n = jd_ref[pl.ds(b * NB + n, 1), :]
        sched_n = jnp.where(jd_n > 0.0, 0.0, tjr_n - jnp.min(tjr_n))
        nr = lax.dot_general(
            j1h, jnp.concatenate([no_n, sched_n], axis=0),
            (((1,), (1,)), ((), ())), preferred_element_type=f32)
        no_r = nr[:, 0:1]
        sched_r = nr[:, 1:2]
        opready = jnp.where(o_row == no_r, sched_r, 0.0) * (1.0 / SCALE)

        pos = o_row + no_r
        g = jnp.where(l_i == C_AVG, avg,
                      jnp.where(l_i == C_NEL, nelig,
                                jnp.where(l_i == C_RDY, opready,
                                          (pos == l_f).astype(f32))))
        ops32 = jnp.dot(g, tab_ref[...], preferred_element_type=f32)
        ops_ref[n] = ops32.reshape(J, OP, D)[:, :O, :]

        nem_n = jnp.sum(pos_mask, axis=0, keepdims=True)
        tmr_n = tmr_ref[pl.ds(b * NB + n, 1), :]
        a_ma_n = (tmr_n - jnp.min(tmr_n)) * (1.0 / SCALE)
        rem_n = jnp.sum(rem_ref[pl.ds(b * NB + n, 1), :])
        frac_n = nem_n * (1.0 / (rem_n + 1e-6))
        mam = jnp.concatenate([a_ma_n, frac_n], axis=0)
        ma_ref[n] = lax.dot_general(mam, wma_ref[...],
                                    (((0,), (0,)), ((), ())),
                                    preferred_element_type=f32)


@functools.partial(jax.jit, static_argnames=())
def kernel(proc_times, next_op, time_job_ready, job_done, time_ma_ready,
           pad_mask, op_scheduled, W_ops, W_ma):
    f32 = jnp.float32
    no_f = next_op.astype(f32)
    jd_f = job_done.astype(f32)
    rem_f = jnp.logical_not(jnp.logical_or(pad_mask, op_scheduled)) \
        .astype(f32).reshape(B, JO)
    wopsT = W_ops.T
    wmaT = W_ma.T

    full = lambda shape: pl.BlockSpec(shape, lambda b: (0,) * len(shape))

    ops, ma = pl.pallas_call(
        _fused_kernel,
        grid=(B // NB,),
        in_specs=[
            pl.BlockSpec((NB, J, O, M), lambda b: (b, 0, 0, 0)),
            full((B, J)),
            full((B, J)),
            full((B, J)),
            full((B, M)),
            full((B, JO)),
            full((3, D)),
            full((2, D)),
        ],
        out_specs=[
            pl.BlockSpec((NB, J, O, D), lambda b: (b, 0, 0, 0)),
            pl.BlockSpec((NB, M, D), lambda b: (b, 0, 0)),
        ],
        out_shape=[
            jax.ShapeDtypeStruct((B, J, O, D), f32),
            jax.ShapeDtypeStruct((B, M, D), f32),
        ],
        scratch_shapes=[
            pltpu.VMEM((P, D), f32),
            pltpu.VMEM((JOP, J), f32),
        ],
    )(proc_times, no_f, time_job_ready, jd_f, time_ma_ready, rem_f, wopsT, wmaT)

    edge = (proc_times * (1.0 / SCALE)).reshape(B, JO, M)
    return ops, ma, edge


# speedup vs baseline: 1.1409x; 1.0469x over previous
import functools
import math

import jax
import jax.numpy as jnp
from jax import lax
from jax.experimental import pallas as pl
from jax.experimental.pallas import tpu as pltpu

B, J, O, M = 128, 40, 25, 64
D = 256
SCALE = 100.0
JO = J * O
OP = 32
JOP = J * OP
P = 64
C_AVG, C_NEL, C_RDY = 61, 62, 63
NB = 4
R = NB * JOP


def _fused_kernel(pt_ref, no_ref, tjr_ref, jd_ref, tmr_ref, rem_ref,
                  wops_ref, wma_ref, ops_ref, ma_ref, tab_ref, j1h_ref):
    f32 = jnp.float32
    b = pl.program_id(0)

    @pl.when(b == 0)
    def _init():
        p_i = lax.broadcasted_iota(jnp.int32, (P, D), 0)
        d_i = lax.broadcasted_iota(jnp.int32, (P, D), 1)
        d_par = (d_i & 1).astype(f32)
        d_even = (d_i - (d_i & 1)).astype(f32)
        ang = (p_i.astype(f32) * jnp.exp(d_even * (-math.log(10000.0) / D))
               + d_par * (math.pi / 2.0))
        pe = jnp.sin(ang)
        w = wops_ref[...]
        pe = jnp.where(p_i == C_AVG, w[0:1, :], pe)
        pe = jnp.where(p_i == C_NEL, w[1:2, :], pe)
        pe = jnp.where(p_i == C_RDY, w[2:3, :], pe)
        tab_ref[...] = pe
        r1 = lax.broadcasted_iota(jnp.int32, (JOP, 1), 0)
        j1h_ref[...] = (lax.broadcasted_iota(jnp.int32, (JOP, J), 1)
                        == (r1 >> 5)).astype(f32)

    pt = pt_ref[...]

    pt2 = jnp.concatenate(
        [pt, jnp.zeros((NB, J, OP - O, M), f32)], axis=2).reshape(R, M)
    pos_mask = (pt2 > 0.0).astype(f32)
    ones_avg = jnp.full((M, 1), 1.0 / (M * SCALE), f32)
    ones_nel = jnp.full((M, 1), 1.0 / M, f32)
    avg = jnp.dot(pt2, ones_avg, preferred_element_type=f32)
    nelig = jnp.dot(pos_mask, ones_nel, preferred_element_type=f32)

    r = lax.broadcasted_iota(jnp.int32, (R, 1), 0)
    o_row = (r & (OP - 1)).astype(f32)
    j1h = j1h_ref[...]
    nr_parts = []
    for n in range(NB):
        no_n = no_ref[pl.ds(b * NB + n, 1), :]
        tjr_n = tjr_ref[pl.ds(b * NB + n, 1), :]
        jd_n = jd_ref[pl.ds(b * NB + n, 1), :]
        sched_n = jnp.where(jd_n > 0.0, 0.0, tjr_n - jnp.min(tjr_n))
        nr_parts.append(lax.dot_general(
            j1h, jnp.concatenate([no_n, sched_n], axis=0),
            (((1,), (1,)), ((), ())), preferred_element_type=f32))
    nr = jnp.concatenate(nr_parts, axis=0)
    no_r = nr[:, 0:1]
    sched_r = nr[:, 1:2]
    opready = jnp.where(o_row == no_r, sched_r, 0.0) * (1.0 / SCALE)

    pos = o_row + no_r
    l_i = lax.broadcasted_iota(jnp.int32, (R, P), 1)
    g = jnp.where(l_i == C_AVG, avg,
                  jnp.where(l_i == C_NEL, nelig,
                            jnp.where(l_i == C_RDY, opready,
                                      (pos == l_i.astype(f32)).astype(f32))))
    ops32 = jnp.dot(g, tab_ref[...], preferred_element_type=f32)
    ops_ref[...] = ops32.reshape(NB * J, OP, D)[:, :O, :] \
        .reshape(NB, J, O, D)

    nem = jnp.sum(pos_mask.reshape(NB, JOP, M), axis=1)
    for n in range(NB):
        tmr_n = tmr_ref[pl.ds(b * NB + n, 1), :]
        a_ma_n = (tmr_n - jnp.min(tmr_n)) * (1.0 / SCALE)
        rem_n = jnp.sum(rem_ref[pl.ds(b * NB + n, 1), :])
        frac_n = nem[n:n + 1, :] * (1.0 / (rem_n + 1e-6))
        mam = jnp.concatenate([a_ma_n, frac_n], axis=0)
        ma_ref[n] = lax.dot_general(mam, wma_ref[...],
                                    (((0,), (0,)), ((), ())),
                                    preferred_element_type=f32)


@functools.partial(jax.jit, static_argnames=())
def kernel(proc_times, next_op, time_job_ready, job_done, time_ma_ready,
           pad_mask, op_scheduled, W_ops, W_ma):
    f32 = jnp.float32
    no_f = next_op.astype(f32)
    jd_f = job_done.astype(f32)
    rem_f = jnp.logical_not(jnp.logical_or(pad_mask, op_scheduled)) \
        .astype(f32).reshape(B, JO)
    wopsT = W_ops.T
    wmaT = W_ma.T

    full = lambda shape: pl.BlockSpec(shape, lambda b: (0,) * len(shape))

    ops, ma = pl.pallas_call(
        _fused_kernel,
        grid=(B // NB,),
        in_specs=[
            pl.BlockSpec((NB, J, O, M), lambda b: (b, 0, 0, 0)),
            full((B, J)),
            full((B, J)),
            full((B, J)),
            full((B, M)),
            full((B, JO)),
            full((3, D)),
            full((2, D)),
        ],
        out_specs=[
            pl.BlockSpec((NB, J, O, D), lambda b: (b, 0, 0, 0)),
            pl.BlockSpec((NB, M, D), lambda b: (b, 0, 0)),
        ],
        out_shape=[
            jax.ShapeDtypeStruct((B, J, O, D), f32),
            jax.ShapeDtypeStruct((B, M, D), f32),
        ],
        scratch_shapes=[
            pltpu.VMEM((P, D), f32),
            pltpu.VMEM((JOP, J), f32),
        ],
    )(proc_times, no_f, time_job_ready, jd_f, time_ma_ready, rem_f, wopsT, wmaT)

    edge = (proc_times * (1.0 / SCALE)).reshape(B, JO, M)
    return ops, ma, edge
